# baseline (speedup 1.0000x reference)
import functools

import jax
import jax.numpy as jnp
from jax import lax
from jax.experimental import pallas as pl
from jax.experimental.pallas import tpu as pltpu
from jax.experimental.pallas import tpu_sc as plsc

N = 10000
E = 320000
D = 128

NC = 2
NS = 16
NW = NC * NS

CH = 96
NCHUNK = 105
EPW = NCHUNK * CH
EPAD = NW * EPW
NPAD = 10112
RPT = NPAD // NS


def _agg_body(x_hbm, src_hbm, dst_hbm, zeros_hbm, out0_hbm, out1_hbm,
              acc, src_v, dst_v, rows0, rows1, g0, g1, isem):
    c = lax.axis_index("c")
    s = lax.axis_index("s")
    wid = s * NC + c

    pltpu.async_copy(src_hbm.at[pl.ds(wid * EPW, EPW)], src_v, isem)
    pltpu.async_copy(dst_hbm.at[wid], dst_v, isem)
    pltpu.sync_copy(zeros_hbm, acc.at[pl.ds(s * RPT, RPT)])
    pltpu.make_async_copy(src_hbm.at[pl.ds(wid * EPW, EPW)], src_v,
                          isem).wait()
    pltpu.make_async_copy(dst_hbm.at[wid], dst_v, isem).wait()
    plsc.subcore_barrier()

    def gather(chunk, rows, sem):
        return pltpu.async_copy(
            x_hbm.at[src_v.at[pl.ds(chunk * CH, CH)]], rows, sem)

    def gwait(rows, sem):
        pltpu.make_async_copy(x_hbm.at[src_v.at[pl.ds(0, CH)]], rows,
                              sem).wait()

    def scat(chunk, rows):
        pltpu.sync_copy(rows, acc.at[dst_v.at[chunk]], add=True)

    gather(0, rows0, g0)

    def body(t, carry):
        c0 = 2 * t
        gwait(rows0, g0)
        gather(c0 + 1, rows1, g1)
        scat(c0, rows0)
        gwait(rows1, g1)
        gather(c0 + 2, rows0, g0)
        scat(c0 + 1, rows1)
        return carry

    lax.fori_loop(0, (NCHUNK - 1) // 2, body, 0)
    gwait(rows0, g0)
    scat(NCHUNK - 1, rows0)

    plsc.subcore_barrier()

    @pl.when(c == 0)
    def _():
        pltpu.sync_copy(acc.at[pl.ds(s * RPT, RPT)],
                        out0_hbm.at[pl.ds(s * RPT, RPT)])

    @pl.when(c == 1)
    def _():
        pltpu.sync_copy(acc.at[pl.ds(s * RPT, RPT)],
                        out1_hbm.at[pl.ds(s * RPT, RPT)])


def _aggregate(x, src, dst3d, zeros_rows):
    mesh = plsc.VectorSubcoreMesh(core_axis_name="c", subcore_axis_name="s")
    f = pl.kernel(
        _agg_body,
        out_type=[jax.ShapeDtypeStruct((NPAD, D), jnp.float32),
                  jax.ShapeDtypeStruct((NPAD, D), jnp.float32)],
        mesh=mesh,
        scratch_types=[
            pltpu.VMEM_SHARED((NPAD, D), jnp.float32),
            pltpu.VMEM((EPW,), jnp.int32),
            pltpu.VMEM((NCHUNK, CH), jnp.int32),
            pltpu.VMEM((CH, D), jnp.float32),
            pltpu.VMEM((CH, D), jnp.float32),
            pltpu.SemaphoreType.DMA,
            pltpu.SemaphoreType.DMA,
            pltpu.SemaphoreType.DMA,
        ],
    )
    return f(x, src, dst3d, zeros_rows)


def _mlp_body(x_ref, p0_ref, p1_ref, scale_ref, wa_ref, ba_ref, wb_ref,
              bb_ref, o_ref, *, final):
    h = (x_ref[...] * scale_ref[...] + p0_ref[...] + p1_ref[...])
    t = lax.dot_general(h, wa_ref[...], (((1,), (1,)), ((), ())),
                        preferred_element_type=jnp.float32)
    t = jnp.maximum(t + ba_ref[...], 0.0)
    z = lax.dot_general(t, wb_ref[...], (((1,), (1,)), ((), ())),
                        preferred_element_type=jnp.float32)
    z = z + bb_ref[...]
    if final:
        m = jnp.max(z, axis=1, keepdims=True)
        e = jnp.exp(z - m)
        lse = jnp.log(jnp.sum(e, axis=1, keepdims=True)) + m
        o_ref[...] = z - lse
    else:
        o_ref[...] = jnp.maximum(z, 0.0)


def _mlp(x, p0, p1, scale, wa, ba, wb, bb, final):
    bn = 1000
    grid = (N // bn,)
    row_spec = pl.BlockSpec((bn, D), lambda i: (i, 0))
    full_spec = pl.BlockSpec((D, D), lambda i: (0, 0))
    vec_spec = pl.BlockSpec((1, D), lambda i: (0, 0))
    return pl.pallas_call(
        functools.partial(_mlp_body, final=final),
        grid=grid,
        in_specs=[row_spec, row_spec, row_spec, vec_spec,
                  full_spec, vec_spec, full_spec, vec_spec],
        out_specs=row_spec,
        out_shape=jax.ShapeDtypeStruct((N, D), jnp.float32),
    )(x, p0, p1, scale, wa, ba, wb, bb)


def kernel(x, edge_index, eps1, W1a, b1a, W1b, b1b, eps2, W2a, b2a, W2b, b2b):
    src = edge_index[0].astype(jnp.int32)
    dst = edge_index[1].astype(jnp.int32)
    npad_e = EPAD - E
    src = jnp.concatenate([src, jnp.zeros((npad_e,), jnp.int32)])
    pad_dst = N + jnp.arange(npad_e, dtype=jnp.int32) % (NPAD - N)
    dst3d = jnp.concatenate([dst, pad_dst]).reshape(NW, NCHUNK, CH)
    zeros_rows = jnp.zeros((RPT, D), jnp.float32)

    p10, p11 = _aggregate(x, src, dst3d, zeros_rows)
    scale1 = jnp.full((1, D), 1.0, jnp.float32) + eps1
    h = _mlp(x, p10, p11, scale1, W1a, b1a.reshape(1, D),
             W1b, b1b.reshape(1, D), final=False)

    p20, p21 = _aggregate(h, src, dst3d, zeros_rows)
    scale2 = jnp.full((1, D), 1.0, jnp.float32) + eps2
    out = _mlp(h, p20, p21, scale2, W2a, b2a.reshape(1, D),
               W2b, b2b.reshape(1, D), final=True)
    return out


# speedup vs baseline: 1.6681x; 1.6681x over previous
import functools

import jax
import jax.numpy as jnp
from jax import lax
from jax.experimental import pallas as pl
from jax.experimental.pallas import tpu as pltpu
from jax.experimental.pallas import tpu_sc as plsc

N = 10000
E = 320000
D = 128

NC = 2
NS = 16
NW = NC * NS

CH = 96
NCHUNK = 105
EPW = NCHUNK * CH
EPAD = NW * EPW
NPAD = 10112
RPT = NPAD // NS


def _agg_body(x_hbm, src_hbm, dst_hbm, zeros_hbm, out0_hbm, out1_hbm,
              acc, src_v, dst_v, rows0, rows1, g0, g1, isem):
    c = lax.axis_index("c")
    s = lax.axis_index("s")
    wid = s * NC + c

    pltpu.async_copy(src_hbm.at[pl.ds(wid * EPW, EPW)], src_v, isem)
    pltpu.async_copy(dst_hbm.at[wid], dst_v, isem)
    pltpu.sync_copy(zeros_hbm, acc.at[pl.ds(s * RPT, RPT)])
    pltpu.make_async_copy(src_hbm.at[pl.ds(wid * EPW, EPW)], src_v,
                          isem).wait()
    pltpu.make_async_copy(dst_hbm.at[wid], dst_v, isem).wait()
    plsc.subcore_barrier()

    def gather(chunk, rows, sem):
        return pltpu.async_copy(
            x_hbm.at[src_v.at[pl.ds(chunk * CH, CH)]], rows, sem)

    def gwait(rows, sem):
        pltpu.make_async_copy(x_hbm.at[src_v.at[pl.ds(0, CH)]], rows,
                              sem).wait()

    def scat(chunk, rows):
        pltpu.sync_copy(rows, acc.at[dst_v.at[chunk]], add=True)

    gather(0, rows0, g0)

    def body(t, carry):
        c0 = 2 * t
        gwait(rows0, g0)
        gather(c0 + 1, rows1, g1)
        scat(c0, rows0)
        gwait(rows1, g1)
        gather(c0 + 2, rows0, g0)
        scat(c0 + 1, rows1)
        return carry

    lax.fori_loop(0, (NCHUNK - 1) // 2, body, 0)
    gwait(rows0, g0)
    scat(NCHUNK - 1, rows0)

    plsc.subcore_barrier()

    @pl.when(c == 0)
    def _():
        pltpu.sync_copy(acc.at[pl.ds(s * RPT, RPT)],
                        out0_hbm.at[pl.ds(s * RPT, RPT)])

    @pl.when(c == 1)
    def _():
        pltpu.sync_copy(acc.at[pl.ds(s * RPT, RPT)],
                        out1_hbm.at[pl.ds(s * RPT, RPT)])


def _aggregate(x, src, dst3d, zeros_rows):
    mesh = plsc.VectorSubcoreMesh(core_axis_name="c", subcore_axis_name="s")
    f = pl.kernel(
        _agg_body,
        out_type=[jax.ShapeDtypeStruct((NPAD, D), jnp.float32),
                  jax.ShapeDtypeStruct((NPAD, D), jnp.float32)],
        mesh=mesh,
        scratch_types=[
            pltpu.VMEM_SHARED((NPAD, D), jnp.float32),
            pltpu.VMEM((EPW,), jnp.int32),
            pltpu.VMEM((NCHUNK, CH), jnp.int32),
            pltpu.VMEM((CH, D), jnp.float32),
            pltpu.VMEM((CH, D), jnp.float32),
            pltpu.SemaphoreType.DMA,
            pltpu.SemaphoreType.DMA,
            pltpu.SemaphoreType.DMA,
        ],
    )
    return f(x, src, dst3d, zeros_rows)


def _mlp_body(x_ref, p0_ref, p1_ref, scale_ref, wa_ref, ba_ref, wb_ref,
              bb_ref, o_ref, *, final):
    h = (x_ref[...] * scale_ref[...] + p0_ref[...] + p1_ref[...])
    t = lax.dot_general(h, wa_ref[...], (((1,), (1,)), ((), ())),
                        preferred_element_type=jnp.float32)
    t = jnp.maximum(t + ba_ref[...], 0.0)
    z = lax.dot_general(t, wb_ref[...], (((1,), (1,)), ((), ())),
                        preferred_element_type=jnp.float32)
    z = z + bb_ref[...]
    if final:
        m = jnp.max(z, axis=1, keepdims=True)
        e = jnp.exp(z - m)
        lse = jnp.log(jnp.sum(e, axis=1, keepdims=True)) + m
        o_ref[...] = z - lse
    else:
        o_ref[...] = jnp.maximum(z, 0.0)


def _mlp(x, p0, p1, scale, wa, ba, wb, bb, final):
    bn = 1000
    grid = (N // bn,)
    row_spec = pl.BlockSpec((bn, D), lambda i: (i, 0))
    full_spec = pl.BlockSpec((D, D), lambda i: (0, 0))
    vec_spec = pl.BlockSpec((1, D), lambda i: (0, 0))
    return pl.pallas_call(
        functools.partial(_mlp_body, final=final),
        grid=grid,
        in_specs=[row_spec, row_spec, row_spec, vec_spec,
                  full_spec, vec_spec, full_spec, vec_spec],
        out_specs=row_spec,
        out_shape=jax.ShapeDtypeStruct((N, D), jnp.float32),
    )(x, p0, p1, scale, wa, ba, wb, bb)


def kernel(x, edge_index, eps1, W1a, b1a, W1b, b1b, eps2, W2a, b2a, W2b, b2b):
    src = edge_index[0].astype(jnp.int32)
    dst = edge_index[1].astype(jnp.int32)
    npad_e = EPAD - E
    pad_iota = jnp.arange(npad_e, dtype=jnp.int32)
    src = jnp.concatenate([src, pad_iota % N])
    pad_dst = N + pad_iota % (NPAD - N)
    dst3d = jnp.concatenate([dst, pad_dst]).reshape(NW, NCHUNK, CH)
    zeros_rows = jnp.zeros((RPT, D), jnp.float32)

    p10, p11 = _aggregate(x, src, dst3d, zeros_rows)
    scale1 = jnp.full((1, D), 1.0, jnp.float32) + eps1
    h = _mlp(x, p10, p11, scale1, W1a, b1a.reshape(1, D),
             W1b, b1b.reshape(1, D), final=False)

    p20, p21 = _aggregate(h, src, dst3d, zeros_rows)
    scale2 = jnp.full((1, D), 1.0, jnp.float32) + eps2
    out = _mlp(h, p20, p21, scale2, W2a, b2a.reshape(1, D),
               W2b, b2b.reshape(1, D), final=True)
    return out


# speedup vs baseline: 1.8486x; 1.1083x over previous
import functools

import jax
import jax.numpy as jnp
from jax import lax
from jax.experimental import pallas as pl
from jax.experimental.pallas import tpu as pltpu
from jax.experimental.pallas import tpu_sc as plsc

N = 10000
E = 320000
D = 128

NC = 2
NS = 16
NW = NC * NS

CH = 128
NCHUNK = 79
EPW = NCHUNK * CH
EPAD = NW * EPW
NPAD = 10112
RPT = NPAD // NS


def _agg_body(x_hbm, src_hbm, dst_hbm, zeros_hbm, out0_hbm, out1_hbm,
              acc, src_v, di0, di1, rows0, rows1,
              g0, g1, d0, d1, isem):
    c = lax.axis_index("c")
    s = lax.axis_index("s")
    wid = s * NC + c

    pltpu.async_copy(src_hbm.at[pl.ds(wid * EPW, EPW)], src_v, isem)
    pltpu.sync_copy(zeros_hbm, acc.at[pl.ds(s * RPT, RPT)])
    pltpu.make_async_copy(src_hbm.at[pl.ds(wid * EPW, EPW)], src_v,
                          isem).wait()
    plsc.subcore_barrier()

    base = wid * EPW

    def gather(chunk, rows, sem):
        return pltpu.async_copy(
            x_hbm.at[src_v.at[pl.ds(chunk * CH, CH)]], rows, sem)

    def gwait(rows, sem):
        pltpu.make_async_copy(x_hbm.at[src_v.at[pl.ds(0, CH)]], rows,
                              sem).wait()

    def dload(chunk, di, sem):
        pltpu.async_copy(dst_hbm.at[pl.ds(base + chunk * CH, CH)], di, sem)

    def dwait(di, sem):
        pltpu.make_async_copy(dst_hbm.at[pl.ds(base, CH)], di, sem).wait()

    def scat(rows, di):
        pltpu.sync_copy(rows, acc.at[di], add=True)

    dload(0, di0, d0)
    gather(0, rows0, g0)

    def body(t, carry):
        c0 = 2 * t
        gwait(rows0, g0)
        dload(c0 + 1, di1, d1)
        gather(c0 + 1, rows1, g1)
        dwait(di0, d0)
        scat(rows0, di0)
        gwait(rows1, g1)
        dload(c0 + 2, di0, d0)
        gather(c0 + 2, rows0, g0)
        dwait(di1, d1)
        scat(rows1, di1)
        return carry

    lax.fori_loop(0, (NCHUNK - 1) // 2, body, 0)
    gwait(rows0, g0)
    dwait(di0, d0)
    scat(rows0, di0)

    plsc.subcore_barrier()

    @pl.when(c == 0)
    def _():
        pltpu.sync_copy(acc.at[pl.ds(s * RPT, RPT)],
                        out0_hbm.at[pl.ds(s * RPT, RPT)])

    @pl.when(c == 1)
    def _():
        pltpu.sync_copy(acc.at[pl.ds(s * RPT, RPT)],
                        out1_hbm.at[pl.ds(s * RPT, RPT)])


def _aggregate(x, src, dst, zeros_rows):
    mesh = plsc.VectorSubcoreMesh(core_axis_name="c", subcore_axis_name="s")
    f = pl.kernel(
        _agg_body,
        out_type=[jax.ShapeDtypeStruct((NPAD, D), jnp.float32),
                  jax.ShapeDtypeStruct((NPAD, D), jnp.float32)],
        mesh=mesh,
        scratch_types=[
            pltpu.VMEM_SHARED((NPAD, D), jnp.float32),
            pltpu.VMEM((EPW,), jnp.int32),
            pltpu.VMEM((CH,), jnp.int32),
            pltpu.VMEM((CH,), jnp.int32),
            pltpu.VMEM((CH, D), jnp.float32),
            pltpu.VMEM((CH, D), jnp.float32),
            pltpu.SemaphoreType.DMA,
            pltpu.SemaphoreType.DMA,
            pltpu.SemaphoreType.DMA,
            pltpu.SemaphoreType.DMA,
            pltpu.SemaphoreType.DMA,
        ],
    )
    return f(x, src, dst, zeros_rows)


def _mlp_body(x_ref, p0_ref, p1_ref, scale_ref, wa_ref, ba_ref, wb_ref,
              bb_ref, o_ref, *, final):
    h = (x_ref[...] * scale_ref[...] + p0_ref[...] + p1_ref[...])
    t = lax.dot_general(h, wa_ref[...], (((1,), (1,)), ((), ())),
                        preferred_element_type=jnp.float32)
    t = jnp.maximum(t + ba_ref[...], 0.0)
    z = lax.dot_general(t, wb_ref[...], (((1,), (1,)), ((), ())),
                        preferred_element_type=jnp.float32)
    z = z + bb_ref[...]
    if final:
        m = jnp.max(z, axis=1, keepdims=True)
        e = jnp.exp(z - m)
        lse = jnp.log(jnp.sum(e, axis=1, keepdims=True)) + m
        o_ref[...] = z - lse
    else:
        o_ref[...] = jnp.maximum(z, 0.0)


def _mlp(x, p0, p1, scale, wa, ba, wb, bb, final):
    bn = 1000
    grid = (N // bn,)
    row_spec = pl.BlockSpec((bn, D), lambda i: (i, 0))
    full_spec = pl.BlockSpec((D, D), lambda i: (0, 0))
    vec_spec = pl.BlockSpec((1, D), lambda i: (0, 0))
    return pl.pallas_call(
        functools.partial(_mlp_body, final=final),
        grid=grid,
        in_specs=[row_spec, row_spec, row_spec, vec_spec,
                  full_spec, vec_spec, full_spec, vec_spec],
        out_specs=row_spec,
        out_shape=jax.ShapeDtypeStruct((N, D), jnp.float32),
    )(x, p0, p1, scale, wa, ba, wb, bb)


def kernel(x, edge_index, eps1, W1a, b1a, W1b, b1b, eps2, W2a, b2a, W2b, b2b):
    src = edge_index[0].astype(jnp.int32)
    dst = edge_index[1].astype(jnp.int32)
    npad_e = EPAD - E
    pad_iota = jnp.arange(npad_e, dtype=jnp.int32)
    src = jnp.concatenate([src, pad_iota % N])
    pad_dst = N + pad_iota % (NPAD - N)
    dst = jnp.concatenate([dst, pad_dst])
    zeros_rows = jnp.zeros((RPT, D), jnp.float32)

    p10, p11 = _aggregate(x, src, dst, zeros_rows)
    scale1 = jnp.full((1, D), 1.0, jnp.float32) + eps1
    h = _mlp(x, p10, p11, scale1, W1a, b1a.reshape(1, D),
             W1b, b1b.reshape(1, D), final=False)

    p20, p21 = _aggregate(h, src, dst, zeros_rows)
    scale2 = jnp.full((1, D), 1.0, jnp.float32) + eps2
    out = _mlp(h, p20, p21, scale2, W2a, b2a.reshape(1, D),
               W2b, b2b.reshape(1, D), final=True)
    return out


# speedup vs baseline: 1.8885x; 1.0216x over previous
import functools

import jax
import jax.numpy as jnp
from jax import lax
from jax.experimental import pallas as pl
from jax.experimental.pallas import tpu as pltpu
from jax.experimental.pallas import tpu_sc as plsc

N = 10000
E = 320000
D = 128

NC = 2
NS = 16
NW = NC * NS

CH = 128
NCHUNK = 79
EPW = NCHUNK * CH
EPAD = NW * EPW
NPAD = 10112
RPT = NPAD // NS


def _agg_body(x_hbm, src_hbm, dst_hbm, zeros_hbm, out0_hbm, out1_hbm,
              acc, src_v, di0, di1, rows0, rows1,
              g0, g1, d0, d1, isem):
    c = lax.axis_index("c")
    s = lax.axis_index("s")
    wid = s * NC + c

    pltpu.async_copy(src_hbm.at[pl.ds(wid * EPW, EPW)], src_v, isem)
    pltpu.sync_copy(zeros_hbm.at[pl.ds(s * RPT, RPT)],
                    acc.at[pl.ds(s * RPT, RPT)])
    pltpu.make_async_copy(src_hbm.at[pl.ds(wid * EPW, EPW)], src_v,
                          isem).wait()
    plsc.subcore_barrier()

    base = wid * EPW

    def gather(chunk, rows, sem):
        return pltpu.async_copy(
            x_hbm.at[src_v.at[pl.ds(chunk * CH, CH)]], rows, sem)

    def gwait(rows, sem):
        pltpu.make_async_copy(x_hbm.at[src_v.at[pl.ds(0, CH)]], rows,
                              sem).wait()

    def dload(chunk, di, sem):
        pltpu.async_copy(dst_hbm.at[pl.ds(base + chunk * CH, CH)], di, sem)

    def dwait(di, sem):
        pltpu.make_async_copy(dst_hbm.at[pl.ds(base, CH)], di, sem).wait()

    def scat(rows, di):
        pltpu.sync_copy(rows, acc.at[di], add=True)

    dload(0, di0, d0)
    gather(0, rows0, g0)

    def body(t, carry):
        c0 = 2 * t
        gwait(rows0, g0)
        dload(c0 + 1, di1, d1)
        gather(c0 + 1, rows1, g1)
        dwait(di0, d0)
        scat(rows0, di0)
        gwait(rows1, g1)
        dload(c0 + 2, di0, d0)
        gather(c0 + 2, rows0, g0)
        dwait(di1, d1)
        scat(rows1, di1)
        return carry

    lax.fori_loop(0, (NCHUNK - 1) // 2, body, 0)
    gwait(rows0, g0)
    dwait(di0, d0)
    scat(rows0, di0)

    plsc.subcore_barrier()

    @pl.when(c == 0)
    def _():
        pltpu.sync_copy(acc.at[pl.ds(s * RPT, RPT)],
                        out0_hbm.at[pl.ds(s * RPT, RPT)])

    @pl.when(c == 1)
    def _():
        pltpu.sync_copy(acc.at[pl.ds(s * RPT, RPT)],
                        out1_hbm.at[pl.ds(s * RPT, RPT)])


def _aggregate(x, src, dst, zeros_rows):
    mesh = plsc.VectorSubcoreMesh(core_axis_name="c", subcore_axis_name="s")
    f = pl.kernel(
        _agg_body,
        out_type=[jax.ShapeDtypeStruct((NPAD, D), jnp.float32),
                  jax.ShapeDtypeStruct((NPAD, D), jnp.float32)],
        mesh=mesh,
        scratch_types=[
            pltpu.VMEM_SHARED((NPAD, D), jnp.float32),
            pltpu.VMEM((EPW,), jnp.int32),
            pltpu.VMEM((CH,), jnp.int32),
            pltpu.VMEM((CH,), jnp.int32),
            pltpu.VMEM((CH, D), jnp.float32),
            pltpu.VMEM((CH, D), jnp.float32),
            pltpu.SemaphoreType.DMA,
            pltpu.SemaphoreType.DMA,
            pltpu.SemaphoreType.DMA,
            pltpu.SemaphoreType.DMA,
            pltpu.SemaphoreType.DMA,
        ],
    )
    return f(x, src, dst, zeros_rows)


def _mlp_body(x_ref, p0_ref, p1_ref, scale_ref, wa_ref, ba_ref, wb_ref,
              bb_ref, o_ref, *, final):
    h = (x_ref[...] * scale_ref[...] + p0_ref[...] + p1_ref[...])
    t = lax.dot_general(h, wa_ref[...], (((1,), (1,)), ((), ())),
                        preferred_element_type=jnp.float32)
    t = jnp.maximum(t + ba_ref[...], 0.0)
    z = lax.dot_general(t, wb_ref[...], (((1,), (1,)), ((), ())),
                        preferred_element_type=jnp.float32)
    z = z + bb_ref[...]
    if final:
        m = jnp.max(z, axis=1, keepdims=True)
        e = jnp.exp(z - m)
        lse = jnp.log(jnp.sum(e, axis=1, keepdims=True)) + m
        o_ref[...] = z - lse
    else:
        o_ref[...] = jnp.maximum(z, 0.0)


def _mlp(x, p0, p1, scale, wa, ba, wb, bb, final):
    bn = 2000
    grid = (N // bn,)
    row_spec = pl.BlockSpec((bn, D), lambda i: (i, 0))
    full_spec = pl.BlockSpec((D, D), lambda i: (0, 0))
    vec_spec = pl.BlockSpec((1, D), lambda i: (0, 0))
    return pl.pallas_call(
        functools.partial(_mlp_body, final=final),
        grid=grid,
        in_specs=[row_spec, row_spec, row_spec, vec_spec,
                  full_spec, vec_spec, full_spec, vec_spec],
        out_specs=row_spec,
        out_shape=jax.ShapeDtypeStruct((N, D), jnp.float32),
    )(x, p0, p1, scale, wa, ba, wb, bb)


def kernel(x, edge_index, eps1, W1a, b1a, W1b, b1b, eps2, W2a, b2a, W2b, b2b):
    src = edge_index[0].astype(jnp.int32)
    dst = edge_index[1].astype(jnp.int32)
    npad_e = EPAD - E
    pad_iota = jnp.arange(npad_e, dtype=jnp.int32)
    src = jnp.concatenate([src, pad_iota % N])
    pad_dst = N + pad_iota % (NPAD - N)
    dst = jnp.concatenate([dst, pad_dst])
    zeros_rows = jnp.zeros((NPAD, D), jnp.float32)

    p10, p11 = _aggregate(x, src, dst, zeros_rows)
    scale1 = jnp.full((1, D), 1.0, jnp.float32) + eps1
    h = _mlp(x, p10, p11, scale1, W1a, b1a.reshape(1, D),
             W1b, b1b.reshape(1, D), final=False)

    p20, p21 = _aggregate(h, src, dst, zeros_rows)
    scale2 = jnp.full((1, D), 1.0, jnp.float32) + eps2
    out = _mlp(h, p20, p21, scale2, W2a, b2a.reshape(1, D),
               W2b, b2b.reshape(1, D), final=True)
    return out


# speedup vs baseline: 2.3271x; 1.2322x over previous
import functools

import jax
import jax.numpy as jnp
from jax import lax
from jax.experimental import pallas as pl
from jax.experimental.pallas import tpu as pltpu
from jax.experimental.pallas import tpu_sc as plsc

N = 10000
E = 320000
D = 128

NC = 2
NS = 16
NW = NC * NS

CH = 96
NCHUNK = 105
EPW = NCHUNK * CH
EPAD = NW * EPW
NPAD = 10112
RPT = NPAD // NS


def _agg_body(x_hbm, src_hbm, dst_hbm, zeros_hbm, out0_hbm, out1_hbm,
              acc, src_v, di, rows, gsem, dsem, ssem, isem):
    c = lax.axis_index("c")
    s = lax.axis_index("s")
    wid = s * NC + c

    pltpu.async_copy(src_hbm.at[pl.ds(wid * EPW, EPW)], src_v, isem)
    pltpu.sync_copy(zeros_hbm.at[pl.ds(s * RPT, RPT)],
                    acc.at[pl.ds(s * RPT, RPT)])
    pltpu.make_async_copy(src_hbm.at[pl.ds(wid * EPW, EPW)], src_v,
                          isem).wait()
    plsc.subcore_barrier()

    base = wid * EPW

    def gather(chunk, b):
        pltpu.async_copy(x_hbm.at[src_v.at[pl.ds(chunk * CH, CH)]],
                         rows[b], gsem[b])

    def gwait(b):
        pltpu.make_async_copy(x_hbm.at[src_v.at[pl.ds(0, CH)]], rows[b],
                              gsem[b]).wait()

    def dload(chunk, b):
        pltpu.async_copy(dst_hbm.at[pl.ds(base + chunk * CH, CH)],
                         di[b], dsem[b])

    def dwait(b):
        pltpu.make_async_copy(dst_hbm.at[pl.ds(base, CH)], di[b],
                              dsem[b]).wait()

    def scat(b):
        pltpu.async_copy(rows[b], acc.at[di[b]], ssem[b], add=True)

    def swait(b):
        pltpu.make_async_copy(rows[b], acc.at[di[b]], ssem[b]).wait()

    dload(0, 0)
    gather(0, 0)
    dload(1, 1)
    gather(1, 1)
    gwait(0)
    dwait(0)
    scat(0)
    dload(2, 2)
    gather(2, 2)

    def step(chunk, b, prefetch):
        gwait(b)
        dwait(b)
        scat(b)
        if prefetch:
            b2 = (b + 2) % 3
            swait(b2)
            dload(chunk + 2, b2)
            gather(chunk + 2, b2)

    def body(u, carry):
        c0 = 3 * u + 1
        step(c0, 1, True)
        step(c0 + 1, 2, True)
        step(c0 + 2, 0, True)
        return carry

    lax.fori_loop(0, (NCHUNK - 3) // 3, body, 0)
    step(NCHUNK - 2, 1, False)
    step(NCHUNK - 1, 2, False)
    swait(0)
    swait(1)
    swait(2)

    plsc.subcore_barrier()

    @pl.when(c == 0)
    def _():
        pltpu.sync_copy(acc.at[pl.ds(s * RPT, RPT)],
                        out0_hbm.at[pl.ds(s * RPT, RPT)])

    @pl.when(c == 1)
    def _():
        pltpu.sync_copy(acc.at[pl.ds(s * RPT, RPT)],
                        out1_hbm.at[pl.ds(s * RPT, RPT)])


def _aggregate(x, src, dst, zeros_rows):
    mesh = plsc.VectorSubcoreMesh(core_axis_name="c", subcore_axis_name="s")
    f = pl.kernel(
        _agg_body,
        out_type=[jax.ShapeDtypeStruct((NPAD, D), jnp.float32),
                  jax.ShapeDtypeStruct((NPAD, D), jnp.float32)],
        mesh=mesh,
        scratch_types=[
            pltpu.VMEM_SHARED((NPAD, D), jnp.float32),
            pltpu.VMEM((EPW,), jnp.int32),
            [pltpu.VMEM((CH,), jnp.int32) for _ in range(3)],
            [pltpu.VMEM((CH, D), jnp.float32) for _ in range(3)],
            [pltpu.SemaphoreType.DMA for _ in range(3)],
            [pltpu.SemaphoreType.DMA for _ in range(3)],
            [pltpu.SemaphoreType.DMA for _ in range(3)],
            pltpu.SemaphoreType.DMA,
        ],
    )
    return f(x, src, dst, zeros_rows)


def _mlp_body(x_ref, p0_ref, p1_ref, scale_ref, wa_ref, ba_ref, wb_ref,
              bb_ref, o_ref, *, final):
    h = (x_ref[...] * scale_ref[...] + p0_ref[...] + p1_ref[...])
    t = lax.dot_general(h, wa_ref[...], (((1,), (1,)), ((), ())),
                        preferred_element_type=jnp.float32)
    t = jnp.maximum(t + ba_ref[...], 0.0)
    z = lax.dot_general(t, wb_ref[...], (((1,), (1,)), ((), ())),
                        preferred_element_type=jnp.float32)
    z = z + bb_ref[...]
    if final:
        m = jnp.max(z, axis=1, keepdims=True)
        e = jnp.exp(z - m)
        lse = jnp.log(jnp.sum(e, axis=1, keepdims=True)) + m
        o_ref[...] = z - lse
    else:
        o_ref[...] = jnp.maximum(z, 0.0)


def _mlp(x, p0, p1, scale, wa, ba, wb, bb, final):
    bn = 2000
    grid = (N // bn,)
    row_spec = pl.BlockSpec((bn, D), lambda i: (i, 0))
    full_spec = pl.BlockSpec((D, D), lambda i: (0, 0))
    vec_spec = pl.BlockSpec((1, D), lambda i: (0, 0))
    return pl.pallas_call(
        functools.partial(_mlp_body, final=final),
        grid=grid,
        in_specs=[row_spec, row_spec, row_spec, vec_spec,
                  full_spec, vec_spec, full_spec, vec_spec],
        out_specs=row_spec,
        out_shape=jax.ShapeDtypeStruct((N, D), jnp.float32),
    )(x, p0, p1, scale, wa, ba, wb, bb)


def kernel(x, edge_index, eps1, W1a, b1a, W1b, b1b, eps2, W2a, b2a, W2b, b2b):
    src = edge_index[0].astype(jnp.int32)
    dst = edge_index[1].astype(jnp.int32)
    npad_e = EPAD - E
    pad_iota = jnp.arange(npad_e, dtype=jnp.int32)
    src = jnp.concatenate([src, pad_iota % N])
    pad_dst = N + pad_iota % (NPAD - N)
    dst = jnp.concatenate([dst, pad_dst])
    zeros_rows = jnp.zeros((NPAD, D), jnp.float32)

    p10, p11 = _aggregate(x, src, dst, zeros_rows)
    scale1 = jnp.full((1, D), 1.0, jnp.float32) + eps1
    h = _mlp(x, p10, p11, scale1, W1a, b1a.reshape(1, D),
             W1b, b1b.reshape(1, D), final=False)

    p20, p21 = _aggregate(h, src, dst, zeros_rows)
    scale2 = jnp.full((1, D), 1.0, jnp.float32) + eps2
    out = _mlp(h, p20, p21, scale2, W2a, b2a.reshape(1, D),
               W2b, b2b.reshape(1, D), final=True)
    return out
